# baseline (speedup 1.0000x reference)
import jax
import jax.numpy as jnp
from jax.experimental import pallas as pl

N = 100000
E = 1600000


def _gat_conv(x, edge_index, W, a_src, a_dst, b, H, C):
    src = edge_index[0]
    dst = edge_index[1]
    h = (x @ W).reshape(-1, H, C)
    alpha_src = jnp.sum(h * a_src[None], axis=-1)
    alpha_dst = jnp.sum(h * a_dst[None], axis=-1)
    e = jax.nn.leaky_relu(alpha_src[src] + alpha_dst[dst], negative_slope=0.2)
    m = jax.ops.segment_max(e, dst, num_segments=N)
    m = jax.lax.stop_gradient(jnp.where(jnp.isfinite(m), m, 0.0))
    ex = jnp.exp(e - m[dst])
    denom = jax.ops.segment_sum(ex, dst, num_segments=N)
    alpha = ex / (denom[dst] + 1e-16)
    out = jax.ops.segment_sum(h[src] * alpha[:, :, None], dst, num_segments=N)
    return out.reshape(-1, H * C) + b


def _fc_body(h_ref, w_ref, b_ref, o_ref):
    o_ref[...] = jnp.dot(h_ref[...], w_ref[...],
                         preferred_element_type=jnp.float32) + b_ref[...]


def kernel(x, edge_index, W1, a_src1, a_dst1, b1, W2, a_src2, a_dst2, b2, fcW, fcb):
    h1 = jax.nn.elu(_gat_conv(x, edge_index, W1, a_src1, a_dst1, b1, 4, 8))
    h2 = jax.nn.elu(_gat_conv(h1, edge_index, W2, a_src2, a_dst2, b2, 8, 8))
    blk = 1000
    y = pl.pallas_call(
        _fc_body,
        grid=(N // blk,),
        in_specs=[
            pl.BlockSpec((blk, 64), lambda i: (i, 0)),
            pl.BlockSpec((64, 1), lambda i: (0, 0)),
            pl.BlockSpec((1,), lambda i: (0,)),
        ],
        out_specs=pl.BlockSpec((blk, 1), lambda i: (i, 0)),
        out_shape=jax.ShapeDtypeStruct((N, 1), jnp.float32),
    )(h2, fcW, fcb)
    return y


# speedup vs baseline: 19.8838x; 19.8838x over previous
import functools

import jax
import jax.numpy as jnp
from jax import lax
from jax.experimental import pallas as pl
from jax.experimental.pallas import tpu as pltpu
from jax.experimental.pallas import tpu_sc as plsc

N = 100000
E = 1600000

NSC = 2
NTILE = 16
CHUNK = 512
SUB = 128
NCHUNK = 196
E_PAD = NTILE * CHUNK * NCHUNK
PER_TILE = CHUNK * NCHUNK

QUART = 25000
NPASS = 2
DUMP = QUART
ACC_ROWS = QUART + 1
AW = 40
SW = 48
PW = 16

ZCH = 48
ZTAIL = ACC_ROWS - ZCH * CHUNK
WOUT = 1568
WLAST = QUART - (NTILE - 1) * WOUT

_BLK = 200
_GRID = N // _BLK


def _take16(v, idx):
    dnums = lax.GatherDimensionNumbers(
        offset_dims=(), collapsed_slice_dims=(0,), start_index_map=(0,))
    return lax.gather(v, idx[:, None], dnums, (1,),
                      mode=lax.GatherScatterMode.PROMISE_IN_BOUNDS)


def _sc_edge_pass(adst_col, mhat_col):
    mesh = plsc.VectorSubcoreMesh(core_axis_name="c", subcore_axis_name="s")

    @functools.partial(
        pl.kernel,
        mesh=mesh,
        compiler_params=pltpu.CompilerParams(use_tc_tiling_on_sc=False),
        out_type=jax.ShapeDtypeStruct((N, AW), jnp.float32),
        scratch_types=[
            pltpu.VMEM((SUB,), jnp.int32),
            pltpu.VMEM((SUB,), jnp.int32),
            pltpu.VMEM((SUB,), jnp.int32),
            pltpu.VMEM((SUB,), jnp.int32),
            pltpu.VMEM((SUB,), jnp.int32),
            pltpu.VMEM((SUB,), jnp.int32),
            pltpu.VMEM((SUB,), jnp.int32),
            pltpu.VMEM((SUB,), jnp.int32),
            pltpu.VMEM((SUB,), jnp.int32),
            pltpu.VMEM((SUB,), jnp.int32),
            pltpu.VMEM((SUB,), jnp.int32),
            pltpu.VMEM((SUB,), jnp.int32),
            pltpu.VMEM((CHUNK, SW), jnp.float32),
            pltpu.VMEM((CHUNK, PW), jnp.float32),
            pltpu.VMEM((CHUNK, AW), jnp.float32),
            pltpu.VMEM_SHARED((ACC_ROWS, AW), jnp.float32),
            pltpu.SemaphoreType.DMA,
        ],
    )
    def edge_kernel(src_hbm, dst_hbm, s_hbm, p_hbm, out_hbm,
                    si0, si1, si2, si3, di0, di1, di2, di3,
                    dl0, dl1, dl2, dl3, srows, prows, msg, acc, sem):
        sis = [si0, si1, si2, si3]
        dis = [di0, di1, di2, di3]
        dls = [dl0, dl1, dl2, dl3]
        cid = lax.axis_index("c")
        sid = lax.axis_index("s")
        iota = lax.iota(jnp.int32, 16)
        zeros16 = jnp.zeros((16,), jnp.float32)

        idx_adst = adst_col + jnp.minimum(iota, 3)
        idx_mhat = mhat_col + jnp.minimum(iota, 3)
        sel01 = jnp.where(iota < 8, 0, 1)
        sel23 = jnp.where(iota < 8, 2, 3)
        selt = jnp.where(iota < 8, 3, jnp.minimum(iota - 8, 3))
        take = _take16

        for rng in range(NPASS):
            lo = (cid * NPASS + rng) * QUART

            def zrow(b, _):
                msg[b, pl.ds(0, 16)] = zeros16
                msg[b, pl.ds(16, 16)] = zeros16
                msg[b, pl.ds(24, 16)] = zeros16
                return _
            lax.fori_loop(0, CHUNK, zrow, None)
            for q in range(ZCH // NTILE):
                pltpu.sync_copy(
                    msg, acc.at[pl.ds((sid * (ZCH // NTILE) + q) * CHUNK,
                                      CHUNK)])

            @pl.when(sid == 0)
            def _():
                pltpu.sync_copy(msg.at[pl.ds(0, ZTAIL)],
                                acc.at[pl.ds(ZCH * CHUNK, ZTAIL)])

            plsc.subcore_barrier()

            def chunk(k, _):
                off = sid * PER_TILE + k * CHUNK
                for j in range(4):
                    pltpu.sync_copy(
                        src_hbm.at[pl.ds(off + j * SUB, SUB)], sis[j])
                    pltpu.sync_copy(
                        dst_hbm.at[pl.ds(off + j * SUB, SUB)], dis[j])
                handles = []
                for j in range(4):
                    handles.append(pltpu.async_copy(
                        s_hbm.at[sis[j]], srows.at[pl.ds(j * SUB, SUB)], sem))
                    handles.append(pltpu.async_copy(
                        p_hbm.at[dis[j]], prows.at[pl.ds(j * SUB, SUB)], sem))

                for j in range(4):
                    for i in range(SUB // 16):
                        d = dis[j][pl.ds(i * 16, 16)]
                        inr = (d >= lo) & (d < lo + QUART)
                        dls[j][pl.ds(i * 16, 16)] = jnp.where(
                            inr, d - lo, jnp.full((16,), DUMP, jnp.int32))
                for h in handles:
                    h.wait()

                def mrow(b, _):
                    pa = prows[b, pl.ds(0, 16)]
                    a_l = srows[b, pl.ds(32, 16)]
                    z = a_l + take(pa, idx_adst)
                    e = jnp.maximum(z, 0.2 * z)
                    ex = jnp.exp(e - take(pa, idx_mhat))
                    s0 = srows[b, pl.ds(0, 16)]
                    s1 = srows[b, pl.ds(16, 16)]
                    st = srows[b, pl.ds(24, 16)]
                    msg[b, pl.ds(0, 16)] = s0 * take(ex, sel01)
                    msg[b, pl.ds(16, 16)] = s1 * take(ex, sel23)
                    ext_t = take(ex, selt)
                    tail = jnp.where(iota < 8, st * ext_t,
                                     jnp.where(iota < 12, ext_t, 0.0))
                    msg[b, pl.ds(24, 16)] = tail
                    return _
                lax.fori_loop(0, CHUNK, mrow, None)

                for j in range(4):
                    pltpu.sync_copy(msg.at[pl.ds(j * SUB, SUB)],
                                    acc.at[dls[j]], add=True)
                return _
            lax.fori_loop(0, NCHUNK, chunk, None)

            plsc.subcore_barrier()

            @pl.when(sid < NTILE - 1)
            def _():
                base = sid * WOUT
                pltpu.sync_copy(acc.at[pl.ds(base, WOUT)],
                                out_hbm.at[pl.ds(lo + base, WOUT)])

            @pl.when(sid == NTILE - 1)
            def _():
                base = (NTILE - 1) * WOUT
                pltpu.sync_copy(acc.at[pl.ds(base, WLAST)],
                                out_hbm.at[pl.ds(lo + base, WLAST)])

            plsc.subcore_barrier()

    return edge_kernel


_sc_l1 = _sc_edge_pass(0, 4)
_sc_l2a = _sc_edge_pass(0, 8)
_sc_l2b = _sc_edge_pass(4, 12)


def _full(i):
    return 0


def _prep1_body(x_ref, w1_ref, asrc_ref, adst_ref, s_ref, ad_ref, gmax_ref,
                scr_ref):
    i = pl.program_id(0)
    h = x_ref[...] * w1_ref[...]
    asrc = jnp.dot(h, asrc_ref[...], preferred_element_type=jnp.float32,
                   precision=lax.Precision.HIGHEST)
    adst = jnp.dot(h, adst_ref[...], preferred_element_type=jnp.float32,
                   precision=lax.Precision.HIGHEST)
    s_ref[...] = jnp.concatenate(
        [h, asrc, jnp.zeros((_BLK, SW - 36), jnp.float32)], axis=1)
    ad_ref[...] = jnp.concatenate(
        [adst, jnp.zeros((_BLK, 12), jnp.float32)], axis=1)

    @pl.when(i == 0)
    def _():
        scr_ref[...] = jnp.full((8, 128), -jnp.inf, jnp.float32)

    blkmax = jnp.max(asrc, axis=0, keepdims=True)
    cur = scr_ref[pl.ds(0, 1), pl.ds(0, 4)]
    scr_ref[pl.ds(0, 1), pl.ds(0, 4)] = jnp.maximum(cur, blkmax)
    gmax_ref[...] = scr_ref[...]


def _mhat_body(nh, ad_ref, gmax_ref, p_ref):
    adst = ad_ref[pl.ds(0, _BLK), pl.ds(0, nh)]
    g = gmax_ref[pl.ds(0, 1), pl.ds(0, nh)]
    z = adst + g
    mhat = jnp.maximum(z, 0.2 * z)
    parts = [adst, mhat]
    if PW - 2 * nh > 0:
        parts.append(jnp.zeros((_BLK, PW - 2 * nh), jnp.float32))
    p_ref[...] = jnp.concatenate(parts, axis=1)


def _norm(acc, b, nheads):
    msg = acc[:, 0:32]
    den = acc[:, 32:32 + 4]
    den = jnp.repeat(den, 8, axis=1)
    return msg / (den + 1e-16)


def _prep2_body(a_ref, b1_ref, w2_ref, asrc_ref, adst_ref,
                sa_ref, sb_ref, ad_ref, gmax_ref, scr_ref):
    i = pl.program_id(0)
    acc = a_ref[...]
    h1 = _norm(acc, None, 4) + b1_ref[...]
    h1 = jnp.where(h1 > 0, h1, jnp.exp(jnp.minimum(h1, 0.0)) - 1.0)
    h2 = jnp.dot(h1, w2_ref[...], preferred_element_type=jnp.float32,
                 precision=lax.Precision.HIGHEST)
    asrc = jnp.dot(h2, asrc_ref[...], preferred_element_type=jnp.float32,
                   precision=lax.Precision.HIGHEST)
    adst = jnp.dot(h2, adst_ref[...], preferred_element_type=jnp.float32,
                   precision=lax.Precision.HIGHEST)
    zpad = jnp.zeros((_BLK, SW - 36), jnp.float32)
    sa_ref[...] = jnp.concatenate([h2[:, 0:32], asrc[:, 0:4], zpad], axis=1)
    sb_ref[...] = jnp.concatenate([h2[:, 32:64], asrc[:, 4:8], zpad], axis=1)
    ad_ref[...] = jnp.concatenate(
        [adst, jnp.zeros((_BLK, 8), jnp.float32)], axis=1)

    @pl.when(i == 0)
    def _():
        scr_ref[...] = jnp.full((8, 128), -jnp.inf, jnp.float32)

    blkmax = jnp.max(asrc, axis=0, keepdims=True)
    cur = scr_ref[pl.ds(0, 1), pl.ds(0, 8)]
    scr_ref[pl.ds(0, 1), pl.ds(0, 8)] = jnp.maximum(cur, blkmax)
    gmax_ref[...] = scr_ref[...]


def _post2_body(aa_ref, ab_ref, b2_ref, fcw_ref, fcb_ref, y_ref):
    ha = _norm(aa_ref[...], None, 4)
    hb = _norm(ab_ref[...], None, 4)
    h2 = jnp.concatenate([ha, hb], axis=1) + b2_ref[...]
    h2 = jnp.where(h2 > 0, h2, jnp.exp(jnp.minimum(h2, 0.0)) - 1.0)
    y_ref[...] = jnp.dot(h2, fcw_ref[...], preferred_element_type=jnp.float32,
                         precision=lax.Precision.HIGHEST) + fcb_ref[...]


def _blockdiag(a):
    H, C = a.shape
    eye = jnp.eye(H, dtype=a.dtype)
    return (a[:, :, None] * eye[:, None, :]).reshape(H * C, H)


def kernel(x, edge_index, W1, a_src1, a_dst1, b1, W2, a_src2, a_dst2, b2,
           fcW, fcb):
    pad = E_PAD - E
    src = jnp.concatenate([edge_index[0], jnp.zeros((pad,), jnp.int32)])
    dst = jnp.concatenate([edge_index[1], jnp.full((pad,), N, jnp.int32)])

    asrc1 = _blockdiag(a_src1)
    adst1 = _blockdiag(a_dst1)
    asrc2 = _blockdiag(a_src2)
    adst2 = _blockdiag(a_dst2)

    s1, ad1, gmax1 = pl.pallas_call(
        _prep1_body,
        grid=(_GRID,),
        in_specs=[
            pl.BlockSpec((_BLK, 1), lambda i: (i, 0)),
            pl.BlockSpec((1, 32), _full2),
            pl.BlockSpec((32, 4), _full2),
            pl.BlockSpec((32, 4), _full2),
        ],
        out_specs=[
            pl.BlockSpec((_BLK, SW), lambda i: (i, 0)),
            pl.BlockSpec((_BLK, 16), lambda i: (i, 0)),
            pl.BlockSpec((8, 128), _full2),
        ],
        out_shape=[
            jax.ShapeDtypeStruct((N, SW), jnp.float32),
            jax.ShapeDtypeStruct((N, 16), jnp.float32),
            jax.ShapeDtypeStruct((8, 128), jnp.float32),
        ],
        scratch_shapes=[pltpu.VMEM((8, 128), jnp.float32)],
    )(x, W1, asrc1, adst1)

    p1 = pl.pallas_call(
        functools.partial(_mhat_body, 4),
        grid=(_GRID,),
        in_specs=[
            pl.BlockSpec((_BLK, 16), lambda i: (i, 0)),
            pl.BlockSpec((8, 128), _full2),
        ],
        out_specs=pl.BlockSpec((_BLK, PW), lambda i: (i, 0)),
        out_shape=jax.ShapeDtypeStruct((N, PW), jnp.float32),
    )(ad1, gmax1)
    p1 = jnp.concatenate([p1, jnp.zeros((8, PW), jnp.float32)], axis=0)

    acc1 = _sc_l1(src, dst, s1, p1)

    s2a, s2b, ad2, gmax2 = pl.pallas_call(
        _prep2_body,
        grid=(_GRID,),
        in_specs=[
            pl.BlockSpec((_BLK, AW), lambda i: (i, 0)),
            pl.BlockSpec((1, 32), _full2),
            pl.BlockSpec((32, 64), _full2),
            pl.BlockSpec((64, 8), _full2),
            pl.BlockSpec((64, 8), _full2),
        ],
        out_specs=[
            pl.BlockSpec((_BLK, SW), lambda i: (i, 0)),
            pl.BlockSpec((_BLK, SW), lambda i: (i, 0)),
            pl.BlockSpec((_BLK, 16), lambda i: (i, 0)),
            pl.BlockSpec((8, 128), _full2),
        ],
        out_shape=[
            jax.ShapeDtypeStruct((N, SW), jnp.float32),
            jax.ShapeDtypeStruct((N, SW), jnp.float32),
            jax.ShapeDtypeStruct((N, 16), jnp.float32),
            jax.ShapeDtypeStruct((8, 128), jnp.float32),
        ],
        scratch_shapes=[pltpu.VMEM((8, 128), jnp.float32)],
    )(acc1, b1.reshape(1, 32), W2, asrc2, adst2)

    p2 = pl.pallas_call(
        functools.partial(_mhat_body, 8),
        grid=(_GRID,),
        in_specs=[
            pl.BlockSpec((_BLK, 16), lambda i: (i, 0)),
            pl.BlockSpec((8, 128), _full2),
        ],
        out_specs=pl.BlockSpec((_BLK, PW), lambda i: (i, 0)),
        out_shape=jax.ShapeDtypeStruct((N, PW), jnp.float32),
    )(ad2, gmax2)
    p2 = jnp.concatenate([p2, jnp.zeros((8, PW), jnp.float32)], axis=0)

    acc2a = _sc_l2a(src, dst, s2a, p2)
    acc2b = _sc_l2b(src, dst, s2b, p2)

    y = pl.pallas_call(
        _post2_body,
        grid=(_GRID,),
        in_specs=[
            pl.BlockSpec((_BLK, AW), lambda i: (i, 0)),
            pl.BlockSpec((_BLK, AW), lambda i: (i, 0)),
            pl.BlockSpec((1, 64), _full2),
            pl.BlockSpec((64, 1), _full2),
            pl.BlockSpec((1, 1), _full2),
        ],
        out_specs=pl.BlockSpec((_BLK, 1), lambda i: (i, 0)),
        out_shape=jax.ShapeDtypeStruct((N, 1), jnp.float32),
    )(acc2a, acc2b, b2.reshape(1, 64), fcW, fcb.reshape(1, 1))
    return y


def _full2(i):
    return (0, 0)


# speedup vs baseline: 24.9147x; 1.2530x over previous
import functools

import jax
import jax.numpy as jnp
from jax import lax
from jax.experimental import pallas as pl
from jax.experimental.pallas import tpu as pltpu
from jax.experimental.pallas import tpu_sc as plsc

N = 100000
E = 1600000

NSC = 2
NTILE = 16
CHUNK = 256
SUB = 128
NSUB = CHUNK // SUB
NCHUNK = 392
E_PAD = NTILE * CHUNK * NCHUNK
PER_TILE = CHUNK * NCHUNK

QUART = 25000
NPASS = 2
DUMP = QUART
ACC_ROWS = QUART + 1
AW = 40
SW = 48
PW = 16

ZCH = 96
ZTAIL = ACC_ROWS - ZCH * CHUNK
WOUT = 1568
WLAST = QUART - (NTILE - 1) * WOUT

_BLK = 200
_GRID = N // _BLK


def _take16(v, idx):
    dnums = lax.GatherDimensionNumbers(
        offset_dims=(), collapsed_slice_dims=(0,), start_index_map=(0,))
    return lax.gather(v, idx[:, None], dnums, (1,),
                      mode=lax.GatherScatterMode.PROMISE_IN_BOUNDS)


def _sc_edge_pass(adst_col, mhat_col):
    mesh = plsc.VectorSubcoreMesh(core_axis_name="c", subcore_axis_name="s")

    @functools.partial(
        pl.kernel,
        mesh=mesh,
        compiler_params=pltpu.CompilerParams(use_tc_tiling_on_sc=False),
        out_type=jax.ShapeDtypeStruct((N, AW), jnp.float32),
        scratch_types=(
            [pltpu.VMEM((SUB,), jnp.int32)] * 12
            + [
                pltpu.VMEM((CHUNK, SW), jnp.float32),
                pltpu.VMEM((CHUNK, SW), jnp.float32),
                pltpu.VMEM((CHUNK, PW), jnp.float32),
                pltpu.VMEM((CHUNK, PW), jnp.float32),
                pltpu.VMEM((CHUNK, AW), jnp.float32),
                pltpu.VMEM((CHUNK, AW), jnp.float32),
                pltpu.VMEM_SHARED((ACC_ROWS, AW), jnp.float32),
            ]
            + [pltpu.SemaphoreType.DMA] * 6
        ),
    )
    def edge_kernel(src_hbm, dst_hbm, s_hbm, p_hbm, out_hbm,
                    si00, si01, si10, si11, di00, di01, di10, di11,
                    dl00, dl01, dl10, dl11, sr0, sr1, pr0, pr1, ms0, ms1,
                    acc, sg0, sg1, ss0, ss1, sx0, sx1):
        si = [[si00, si01], [si10, si11]]
        di = [[di00, di01], [di10, di11]]
        dl = [[dl00, dl01], [dl10, dl11]]
        srows = [sr0, sr1]
        prows = [pr0, pr1]
        msg = [ms0, ms1]
        sem_g = [sg0, sg1]
        sem_s = [ss0, ss1]
        sem_i = [sx0, sx1]
        cid = lax.axis_index("c")
        sid = lax.axis_index("s")
        iota = lax.iota(jnp.int32, 16)
        zeros16 = jnp.zeros((16,), jnp.float32)

        idx_adst = adst_col + jnp.minimum(iota, 3)
        idx_mhat = mhat_col + jnp.minimum(iota, 3)
        sel01 = jnp.where(iota < 8, 0, 1)
        sel23 = jnp.where(iota < 8, 2, 3)
        selt = jnp.where(iota < 8, 3, jnp.minimum(iota - 8, 3))
        take = _take16

        def idx_copies(buf, k):
            off = sid * PER_TILE + k * CHUNK
            return [(src_hbm.at[pl.ds(off + j * SUB, SUB)], si[buf][j])
                    for j in range(NSUB)] + \
                   [(dst_hbm.at[pl.ds(off + j * SUB, SUB)], di[buf][j])
                    for j in range(NSUB)]

        def gather_copies(buf):
            return [(s_hbm.at[si[buf][j]], srows[buf].at[pl.ds(j * SUB, SUB)])
                    for j in range(NSUB)] + \
                   [(p_hbm.at[di[buf][j]], prows[buf].at[pl.ds(j * SUB, SUB)])
                    for j in range(NSUB)]

        def scatter_copies(buf):
            return [(msg[buf].at[pl.ds(j * SUB, SUB)], acc.at[dl[buf][j]])
                    for j in range(NSUB)]

        def fire(copies, sem, add=False):
            for s, d in copies:
                pltpu.async_copy(s, d, sem, add=add)

        def drain(copies, sem):
            for s, d in copies:
                pltpu.make_async_copy(s, d, sem).wait()

        for rng in range(NPASS):
            lo = (cid * NPASS + rng) * QUART

            def zrow(b, _):
                ms0[b, pl.ds(0, 16)] = zeros16
                ms0[b, pl.ds(16, 16)] = zeros16
                ms0[b, pl.ds(24, 16)] = zeros16
                return _
            lax.fori_loop(0, CHUNK, zrow, None)
            for q in range(ZCH // NTILE):
                pltpu.sync_copy(
                    ms0, acc.at[pl.ds((sid * (ZCH // NTILE) + q) * CHUNK,
                                      CHUNK)])

            @pl.when(sid == 0)
            def _():
                pltpu.sync_copy(ms0.at[pl.ds(0, CHUNK)],
                                acc.at[pl.ds(ZCH * CHUNK, CHUNK)])

            @pl.when(sid == 1)
            def _():
                pltpu.sync_copy(ms0.at[pl.ds(0, ZTAIL - CHUNK)],
                                acc.at[pl.ds((ZCH + 1) * CHUNK,
                                             ZTAIL - CHUNK)])

            plsc.subcore_barrier()

            fire(idx_copies(0, 0), sem_i[0])
            drain(idx_copies(0, 0), sem_i[0])
            fire(gather_copies(0), sem_g[0])

            def pair(k2, _):
                for b in range(2):
                    k = 2 * k2 + b
                    nxt = 1 - b

                    @pl.when(k + 1 < NCHUNK)
                    def _():
                        fire(idx_copies(nxt, k + 1), sem_i[nxt])

                    drain(gather_copies(b), sem_g[b])

                    @pl.when(k >= 2)
                    def _():
                        drain(scatter_copies(b), sem_s[b])

                    for j in range(NSUB):
                        for i in range(SUB // 16):
                            d = di[b][j][pl.ds(i * 16, 16)]
                            inr = (d >= lo) & (d < lo + QUART)
                            dl[b][j][pl.ds(i * 16, 16)] = jnp.where(
                                inr, d - lo, jnp.full((16,), DUMP, jnp.int32))

                    def mrow(i, _):
                        for u in range(4):
                            e_ = i * 4 + u
                            pa = prows[b][e_, pl.ds(0, 16)]
                            a_l = srows[b][e_, pl.ds(32, 16)]
                            z = a_l + take(pa, idx_adst)
                            ee = jnp.maximum(z, 0.2 * z)
                            ex = jnp.exp(ee - take(pa, idx_mhat))
                            s0 = srows[b][e_, pl.ds(0, 16)]
                            s1 = srows[b][e_, pl.ds(16, 16)]
                            st = srows[b][e_, pl.ds(24, 16)]
                            msg[b][e_, pl.ds(0, 16)] = s0 * take(ex, sel01)
                            msg[b][e_, pl.ds(16, 16)] = s1 * take(ex, sel23)
                            ext_t = take(ex, selt)
                            tail = jnp.where(iota < 8, st * ext_t,
                                             jnp.where(iota < 12, ext_t, 0.0))
                            msg[b][e_, pl.ds(24, 16)] = tail
                        return _
                    lax.fori_loop(0, CHUNK // 4, mrow, None)

                    @pl.when(k + 1 < NCHUNK)
                    def _():
                        drain(idx_copies(nxt, k + 1), sem_i[nxt])
                        fire(gather_copies(nxt), sem_g[nxt])

                    fire(scatter_copies(b), sem_s[b], add=True)
                return _
            lax.fori_loop(0, NCHUNK // 2, pair, None)

            drain(scatter_copies(0), sem_s[0])
            drain(scatter_copies(1), sem_s[1])
            plsc.subcore_barrier()

            @pl.when(sid < NTILE - 1)
            def _():
                base = sid * WOUT
                pltpu.sync_copy(acc.at[pl.ds(base, WOUT)],
                                out_hbm.at[pl.ds(lo + base, WOUT)])

            @pl.when(sid == NTILE - 1)
            def _():
                base = (NTILE - 1) * WOUT
                pltpu.sync_copy(acc.at[pl.ds(base, WLAST)],
                                out_hbm.at[pl.ds(lo + base, WLAST)])

            plsc.subcore_barrier()

    return edge_kernel


_sc_l1 = _sc_edge_pass(0, 4)
_sc_l2a = _sc_edge_pass(0, 8)
_sc_l2b = _sc_edge_pass(4, 12)


def _full(i):
    return 0


def _prep1_body(x_ref, w1_ref, asrc_ref, adst_ref, s_ref, ad_ref, gmax_ref,
                scr_ref):
    i = pl.program_id(0)
    h = x_ref[...] * w1_ref[...]
    asrc = jnp.dot(h, asrc_ref[...], preferred_element_type=jnp.float32,
                   precision=lax.Precision.HIGHEST)
    adst = jnp.dot(h, adst_ref[...], preferred_element_type=jnp.float32,
                   precision=lax.Precision.HIGHEST)
    s_ref[...] = jnp.concatenate(
        [h, asrc, jnp.zeros((_BLK, SW - 36), jnp.float32)], axis=1)
    ad_ref[...] = jnp.concatenate(
        [adst, jnp.zeros((_BLK, 12), jnp.float32)], axis=1)

    @pl.when(i == 0)
    def _():
        scr_ref[...] = jnp.full((8, 128), -jnp.inf, jnp.float32)

    blkmax = jnp.max(asrc, axis=0, keepdims=True)
    cur = scr_ref[pl.ds(0, 1), pl.ds(0, 4)]
    scr_ref[pl.ds(0, 1), pl.ds(0, 4)] = jnp.maximum(cur, blkmax)
    gmax_ref[...] = scr_ref[...]


def _mhat_body(nh, ad_ref, gmax_ref, p_ref):
    adst = ad_ref[pl.ds(0, _BLK), pl.ds(0, nh)]
    g = gmax_ref[pl.ds(0, 1), pl.ds(0, nh)]
    z = adst + g
    mhat = jnp.maximum(z, 0.2 * z)
    parts = [adst, mhat]
    if PW - 2 * nh > 0:
        parts.append(jnp.zeros((_BLK, PW - 2 * nh), jnp.float32))
    p_ref[...] = jnp.concatenate(parts, axis=1)


def _norm(acc, b, nheads):
    msg = acc[:, 0:32]
    den = acc[:, 32:32 + 4]
    den = jnp.repeat(den, 8, axis=1)
    return msg / (den + 1e-16)


def _prep2_body(a_ref, b1_ref, w2_ref, asrc_ref, adst_ref,
                sa_ref, sb_ref, ad_ref, gmax_ref, scr_ref):
    i = pl.program_id(0)
    acc = a_ref[...]
    h1 = _norm(acc, None, 4) + b1_ref[...]
    h1 = jnp.where(h1 > 0, h1, jnp.exp(jnp.minimum(h1, 0.0)) - 1.0)
    h2 = jnp.dot(h1, w2_ref[...], preferred_element_type=jnp.float32,
                 precision=lax.Precision.HIGHEST)
    asrc = jnp.dot(h2, asrc_ref[...], preferred_element_type=jnp.float32,
                   precision=lax.Precision.HIGHEST)
    adst = jnp.dot(h2, adst_ref[...], preferred_element_type=jnp.float32,
                   precision=lax.Precision.HIGHEST)
    zpad = jnp.zeros((_BLK, SW - 36), jnp.float32)
    sa_ref[...] = jnp.concatenate([h2[:, 0:32], asrc[:, 0:4], zpad], axis=1)
    sb_ref[...] = jnp.concatenate([h2[:, 32:64], asrc[:, 4:8], zpad], axis=1)
    ad_ref[...] = jnp.concatenate(
        [adst, jnp.zeros((_BLK, 8), jnp.float32)], axis=1)

    @pl.when(i == 0)
    def _():
        scr_ref[...] = jnp.full((8, 128), -jnp.inf, jnp.float32)

    blkmax = jnp.max(asrc, axis=0, keepdims=True)
    cur = scr_ref[pl.ds(0, 1), pl.ds(0, 8)]
    scr_ref[pl.ds(0, 1), pl.ds(0, 8)] = jnp.maximum(cur, blkmax)
    gmax_ref[...] = scr_ref[...]


def _post2_body(aa_ref, ab_ref, b2_ref, fcw_ref, fcb_ref, y_ref):
    ha = _norm(aa_ref[...], None, 4)
    hb = _norm(ab_ref[...], None, 4)
    h2 = jnp.concatenate([ha, hb], axis=1) + b2_ref[...]
    h2 = jnp.where(h2 > 0, h2, jnp.exp(jnp.minimum(h2, 0.0)) - 1.0)
    y_ref[...] = jnp.dot(h2, fcw_ref[...], preferred_element_type=jnp.float32,
                         precision=lax.Precision.HIGHEST) + fcb_ref[...]


def _blockdiag(a):
    H, C = a.shape
    eye = jnp.eye(H, dtype=a.dtype)
    return (a[:, :, None] * eye[:, None, :]).reshape(H * C, H)


def kernel(x, edge_index, W1, a_src1, a_dst1, b1, W2, a_src2, a_dst2, b2,
           fcW, fcb):
    pad = E_PAD - E
    src = jnp.concatenate([edge_index[0], jnp.zeros((pad,), jnp.int32)])
    dst = jnp.concatenate([edge_index[1], jnp.full((pad,), N, jnp.int32)])

    asrc1 = _blockdiag(a_src1)
    adst1 = _blockdiag(a_dst1)
    asrc2 = _blockdiag(a_src2)
    adst2 = _blockdiag(a_dst2)

    s1, ad1, gmax1 = pl.pallas_call(
        _prep1_body,
        grid=(_GRID,),
        in_specs=[
            pl.BlockSpec((_BLK, 1), lambda i: (i, 0)),
            pl.BlockSpec((1, 32), _full2),
            pl.BlockSpec((32, 4), _full2),
            pl.BlockSpec((32, 4), _full2),
        ],
        out_specs=[
            pl.BlockSpec((_BLK, SW), lambda i: (i, 0)),
            pl.BlockSpec((_BLK, 16), lambda i: (i, 0)),
            pl.BlockSpec((8, 128), _full2),
        ],
        out_shape=[
            jax.ShapeDtypeStruct((N, SW), jnp.float32),
            jax.ShapeDtypeStruct((N, 16), jnp.float32),
            jax.ShapeDtypeStruct((8, 128), jnp.float32),
        ],
        scratch_shapes=[pltpu.VMEM((8, 128), jnp.float32)],
    )(x, W1, asrc1, adst1)

    p1 = pl.pallas_call(
        functools.partial(_mhat_body, 4),
        grid=(_GRID,),
        in_specs=[
            pl.BlockSpec((_BLK, 16), lambda i: (i, 0)),
            pl.BlockSpec((8, 128), _full2),
        ],
        out_specs=pl.BlockSpec((_BLK, PW), lambda i: (i, 0)),
        out_shape=jax.ShapeDtypeStruct((N, PW), jnp.float32),
    )(ad1, gmax1)
    p1 = jnp.concatenate([p1, jnp.zeros((8, PW), jnp.float32)], axis=0)

    acc1 = _sc_l1(src, dst, s1, p1)

    s2a, s2b, ad2, gmax2 = pl.pallas_call(
        _prep2_body,
        grid=(_GRID,),
        in_specs=[
            pl.BlockSpec((_BLK, AW), lambda i: (i, 0)),
            pl.BlockSpec((1, 32), _full2),
            pl.BlockSpec((32, 64), _full2),
            pl.BlockSpec((64, 8), _full2),
            pl.BlockSpec((64, 8), _full2),
        ],
        out_specs=[
            pl.BlockSpec((_BLK, SW), lambda i: (i, 0)),
            pl.BlockSpec((_BLK, SW), lambda i: (i, 0)),
            pl.BlockSpec((_BLK, 16), lambda i: (i, 0)),
            pl.BlockSpec((8, 128), _full2),
        ],
        out_shape=[
            jax.ShapeDtypeStruct((N, SW), jnp.float32),
            jax.ShapeDtypeStruct((N, SW), jnp.float32),
            jax.ShapeDtypeStruct((N, 16), jnp.float32),
            jax.ShapeDtypeStruct((8, 128), jnp.float32),
        ],
        scratch_shapes=[pltpu.VMEM((8, 128), jnp.float32)],
    )(acc1, b1.reshape(1, 32), W2, asrc2, adst2)

    p2 = pl.pallas_call(
        functools.partial(_mhat_body, 8),
        grid=(_GRID,),
        in_specs=[
            pl.BlockSpec((_BLK, 16), lambda i: (i, 0)),
            pl.BlockSpec((8, 128), _full2),
        ],
        out_specs=pl.BlockSpec((_BLK, PW), lambda i: (i, 0)),
        out_shape=jax.ShapeDtypeStruct((N, PW), jnp.float32),
    )(ad2, gmax2)
    p2 = jnp.concatenate([p2, jnp.zeros((8, PW), jnp.float32)], axis=0)

    acc2a = _sc_l2a(src, dst, s2a, p2)
    acc2b = _sc_l2b(src, dst, s2b, p2)

    y = pl.pallas_call(
        _post2_body,
        grid=(_GRID,),
        in_specs=[
            pl.BlockSpec((_BLK, AW), lambda i: (i, 0)),
            pl.BlockSpec((_BLK, AW), lambda i: (i, 0)),
            pl.BlockSpec((1, 64), _full2),
            pl.BlockSpec((64, 1), _full2),
            pl.BlockSpec((1, 1), _full2),
        ],
        out_specs=pl.BlockSpec((_BLK, 1), lambda i: (i, 0)),
        out_shape=jax.ShapeDtypeStruct((N, 1), jnp.float32),
    )(acc2a, acc2b, b2.reshape(1, 64), fcW, fcb.reshape(1, 1))
    return y


def _full2(i):
    return (0, 0)


# speedup vs baseline: 38.4109x; 1.5417x over previous
import functools

import jax
import jax.numpy as jnp
from jax import lax
from jax.experimental import pallas as pl
from jax.experimental.pallas import tpu as pltpu
from jax.experimental.pallas import tpu_sc as plsc

N = 100000
E = 1600000

NSC = 2
NTILE = 16
CHUNK = 256
SUB = 128
NSUB = CHUNK // SUB
NCHUNK = 392
E_PAD = NTILE * CHUNK * NCHUNK
PER_TILE = CHUNK * NCHUNK

QUART = 25000
NPASS = 2
DUMP = QUART
ACC_ROWS = QUART + 1
AW = 40
SW = 48
PW = 16

ZCH = 96
ZTAIL = ACC_ROWS - ZCH * CHUNK
WOUT = 1568
WLAST = QUART - (NTILE - 1) * WOUT

_BLK = 200
_GRID = N // _BLK


def _take16(v, idx):
    dnums = lax.GatherDimensionNumbers(
        offset_dims=(), collapsed_slice_dims=(0,), start_index_map=(0,))
    return lax.gather(v, idx[:, None], dnums, (1,),
                      mode=lax.GatherScatterMode.PROMISE_IN_BOUNDS)


def _sc_edge_pass(adst_col, mhat_col):
    mesh = plsc.VectorSubcoreMesh(core_axis_name="c", subcore_axis_name="s")

    @functools.partial(
        pl.kernel,
        mesh=mesh,
        compiler_params=pltpu.CompilerParams(use_tc_tiling_on_sc=False),
        out_type=jax.ShapeDtypeStruct((N, AW), jnp.float32),
        scratch_types=(
            [pltpu.VMEM((SUB,), jnp.int32)] * 12
            + [
                pltpu.VMEM((CHUNK, SW), jnp.float32),
                pltpu.VMEM((CHUNK, SW), jnp.float32),
                pltpu.VMEM((CHUNK, PW), jnp.float32),
                pltpu.VMEM((CHUNK, PW), jnp.float32),
                pltpu.VMEM((CHUNK, AW), jnp.float32),
                pltpu.VMEM((CHUNK, AW), jnp.float32),
                pltpu.VMEM_SHARED((ACC_ROWS, AW), jnp.float32),
            ]
            + [pltpu.SemaphoreType.DMA] * 6
        ),
    )
    def edge_kernel(src_hbm, dst_hbm, s_hbm, p_hbm, out_hbm,
                    si00, si01, si10, si11, di00, di01, di10, di11,
                    dl00, dl01, dl10, dl11, sr0, sr1, pr0, pr1, ms0, ms1,
                    acc, sg0, sg1, ss0, ss1, sx0, sx1):
        si = [[si00, si01], [si10, si11]]
        di = [[di00, di01], [di10, di11]]
        dl = [[dl00, dl01], [dl10, dl11]]
        srows = [sr0, sr1]
        prows = [pr0, pr1]
        msg = [ms0, ms1]
        sem_g = [sg0, sg1]
        sem_s = [ss0, ss1]
        sem_i = [sx0, sx1]
        cid = lax.axis_index("c")
        sid = lax.axis_index("s")
        iota = lax.iota(jnp.int32, 16)
        zeros16 = jnp.zeros((16,), jnp.float32)

        idx_adst = adst_col + jnp.minimum(iota, 3)
        idx_mhat = mhat_col + jnp.minimum(iota, 3)
        sel01 = jnp.where(iota < 8, 0, 1)
        sel23 = jnp.where(iota < 8, 2, 3)
        selt = jnp.where(iota < 8, 3, jnp.minimum(iota - 8, 3))
        take = _take16

        def idx_copies(buf, k):
            off = sid * PER_TILE + k * CHUNK
            return [(src_hbm.at[pl.ds(off + j * SUB, SUB)], si[buf][j])
                    for j in range(NSUB)] + \
                   [(dst_hbm.at[pl.ds(off + j * SUB, SUB)], di[buf][j])
                    for j in range(NSUB)]

        def gather_copies(buf):
            return [(s_hbm.at[si[buf][j]], srows[buf].at[pl.ds(j * SUB, SUB)])
                    for j in range(NSUB)] + \
                   [(p_hbm.at[di[buf][j]], prows[buf].at[pl.ds(j * SUB, SUB)])
                    for j in range(NSUB)]

        def scatter_copies(buf):
            return [(msg[buf].at[pl.ds(j * SUB, SUB)], acc.at[dl[buf][j]])
                    for j in range(NSUB)]

        def fire(copies, sem, add=False):
            for s, d in copies:
                pltpu.async_copy(s, d, sem, add=add)

        def drain(copies, sem):
            for s, d in copies:
                pltpu.make_async_copy(s, d, sem).wait()

        for rng in range(NPASS):
            lo = (cid * NPASS + rng) * QUART

            @plsc.parallel_loop(0, CHUNK, unroll=4)
            def zrow(zb):
                ms0[zb, pl.ds(0, 16)] = zeros16
                ms0[zb, pl.ds(16, 16)] = zeros16
                ms0[zb, pl.ds(24, 16)] = zeros16
            for q in range(ZCH // NTILE):
                pltpu.sync_copy(
                    ms0, acc.at[pl.ds((sid * (ZCH // NTILE) + q) * CHUNK,
                                      CHUNK)])

            @pl.when(sid == 0)
            def _():
                pltpu.sync_copy(ms0.at[pl.ds(0, CHUNK)],
                                acc.at[pl.ds(ZCH * CHUNK, CHUNK)])

            @pl.when(sid == 1)
            def _():
                pltpu.sync_copy(ms0.at[pl.ds(0, ZTAIL - CHUNK)],
                                acc.at[pl.ds((ZCH + 1) * CHUNK,
                                             ZTAIL - CHUNK)])

            plsc.subcore_barrier()

            fire(idx_copies(0, 0), sem_i[0])
            drain(idx_copies(0, 0), sem_i[0])
            fire(gather_copies(0), sem_g[0])

            def pair(k2, _):
                for b in range(2):
                    k = 2 * k2 + b
                    nxt = 1 - b

                    @pl.when(k + 1 < NCHUNK)
                    def _():
                        fire(idx_copies(nxt, k + 1), sem_i[nxt])

                    drain(gather_copies(b), sem_g[b])

                    @pl.when(k >= 2)
                    def _():
                        drain(scatter_copies(b), sem_s[b])

                    for j in range(NSUB):
                        for i in range(SUB // 16):
                            d = di[b][j][pl.ds(i * 16, 16)]
                            inr = (d >= lo) & (d < lo + QUART)
                            dl[b][j][pl.ds(i * 16, 16)] = jnp.where(
                                inr, d - lo, jnp.full((16,), DUMP, jnp.int32))

                    srows_b, prows_b, msg_b = srows[b], prows[b], msg[b]

                    @plsc.parallel_loop(0, CHUNK, unroll=4)
                    def mrow(e_):
                        pa = prows_b[e_, pl.ds(0, 16)]
                        a_l = srows_b[e_, pl.ds(32, 16)]
                        z = a_l + take(pa, idx_adst)
                        ee = jnp.maximum(z, 0.2 * z)
                        ex = jnp.exp(ee - take(pa, idx_mhat))
                        s0 = srows_b[e_, pl.ds(0, 16)]
                        s1 = srows_b[e_, pl.ds(16, 16)]
                        st = srows_b[e_, pl.ds(24, 16)]
                        msg_b[e_, pl.ds(0, 16)] = s0 * take(ex, sel01)
                        msg_b[e_, pl.ds(16, 16)] = s1 * take(ex, sel23)
                        ext_t = take(ex, selt)
                        tail = jnp.where(iota < 8, st * ext_t,
                                         jnp.where(iota < 12, ext_t, 0.0))
                        msg_b[e_, pl.ds(24, 16)] = tail

                    @pl.when(k + 1 < NCHUNK)
                    def _():
                        drain(idx_copies(nxt, k + 1), sem_i[nxt])
                        fire(gather_copies(nxt), sem_g[nxt])

                    fire(scatter_copies(b), sem_s[b], add=True)
                return _
            lax.fori_loop(0, NCHUNK // 2, pair, None)

            drain(scatter_copies(0), sem_s[0])
            drain(scatter_copies(1), sem_s[1])
            plsc.subcore_barrier()

            @pl.when(sid < NTILE - 1)
            def _():
                base = sid * WOUT
                pltpu.sync_copy(acc.at[pl.ds(base, WOUT)],
                                out_hbm.at[pl.ds(lo + base, WOUT)])

            @pl.when(sid == NTILE - 1)
            def _():
                base = (NTILE - 1) * WOUT
                pltpu.sync_copy(acc.at[pl.ds(base, WLAST)],
                                out_hbm.at[pl.ds(lo + base, WLAST)])

            plsc.subcore_barrier()

    return edge_kernel


_sc_l1 = _sc_edge_pass(0, 4)
_sc_l2a = _sc_edge_pass(0, 8)
_sc_l2b = _sc_edge_pass(4, 12)


def _full(i):
    return 0


def _prep1_body(x_ref, w1_ref, asrc_ref, adst_ref, s_ref, ad_ref, gmax_ref,
                scr_ref):
    i = pl.program_id(0)
    h = x_ref[...] * w1_ref[...]
    asrc = jnp.dot(h, asrc_ref[...], preferred_element_type=jnp.float32,
                   precision=lax.Precision.HIGHEST)
    adst = jnp.dot(h, adst_ref[...], preferred_element_type=jnp.float32,
                   precision=lax.Precision.HIGHEST)
    s_ref[...] = jnp.concatenate(
        [h, asrc, jnp.zeros((_BLK, SW - 36), jnp.float32)], axis=1)
    ad_ref[...] = jnp.concatenate(
        [adst, jnp.zeros((_BLK, 12), jnp.float32)], axis=1)

    @pl.when(i == 0)
    def _():
        scr_ref[...] = jnp.full((8, 128), -jnp.inf, jnp.float32)

    blkmax = jnp.max(asrc, axis=0, keepdims=True)
    cur = scr_ref[pl.ds(0, 1), pl.ds(0, 4)]
    scr_ref[pl.ds(0, 1), pl.ds(0, 4)] = jnp.maximum(cur, blkmax)
    gmax_ref[...] = scr_ref[...]


def _mhat_body(nh, ad_ref, gmax_ref, p_ref):
    adst = ad_ref[pl.ds(0, _BLK), pl.ds(0, nh)]
    g = gmax_ref[pl.ds(0, 1), pl.ds(0, nh)]
    z = adst + g
    mhat = jnp.maximum(z, 0.2 * z)
    parts = [adst, mhat]
    if PW - 2 * nh > 0:
        parts.append(jnp.zeros((_BLK, PW - 2 * nh), jnp.float32))
    p_ref[...] = jnp.concatenate(parts, axis=1)


def _norm(acc, b, nheads):
    msg = acc[:, 0:32]
    den = acc[:, 32:32 + 4]
    den = jnp.repeat(den, 8, axis=1)
    return msg / (den + 1e-16)


def _prep2_body(a_ref, b1_ref, w2_ref, asrc_ref, adst_ref,
                sa_ref, sb_ref, ad_ref, gmax_ref, scr_ref):
    i = pl.program_id(0)
    acc = a_ref[...]
    h1 = _norm(acc, None, 4) + b1_ref[...]
    h1 = jnp.where(h1 > 0, h1, jnp.exp(jnp.minimum(h1, 0.0)) - 1.0)
    h2 = jnp.dot(h1, w2_ref[...], preferred_element_type=jnp.float32,
                 precision=lax.Precision.HIGHEST)
    asrc = jnp.dot(h2, asrc_ref[...], preferred_element_type=jnp.float32,
                   precision=lax.Precision.HIGHEST)
    adst = jnp.dot(h2, adst_ref[...], preferred_element_type=jnp.float32,
                   precision=lax.Precision.HIGHEST)
    zpad = jnp.zeros((_BLK, SW - 36), jnp.float32)
    sa_ref[...] = jnp.concatenate([h2[:, 0:32], asrc[:, 0:4], zpad], axis=1)
    sb_ref[...] = jnp.concatenate([h2[:, 32:64], asrc[:, 4:8], zpad], axis=1)
    ad_ref[...] = jnp.concatenate(
        [adst, jnp.zeros((_BLK, 8), jnp.float32)], axis=1)

    @pl.when(i == 0)
    def _():
        scr_ref[...] = jnp.full((8, 128), -jnp.inf, jnp.float32)

    blkmax = jnp.max(asrc, axis=0, keepdims=True)
    cur = scr_ref[pl.ds(0, 1), pl.ds(0, 8)]
    scr_ref[pl.ds(0, 1), pl.ds(0, 8)] = jnp.maximum(cur, blkmax)
    gmax_ref[...] = scr_ref[...]


def _post2_body(aa_ref, ab_ref, b2_ref, fcw_ref, fcb_ref, y_ref):
    ha = _norm(aa_ref[...], None, 4)
    hb = _norm(ab_ref[...], None, 4)
    h2 = jnp.concatenate([ha, hb], axis=1) + b2_ref[...]
    h2 = jnp.where(h2 > 0, h2, jnp.exp(jnp.minimum(h2, 0.0)) - 1.0)
    y_ref[...] = jnp.dot(h2, fcw_ref[...], preferred_element_type=jnp.float32,
                         precision=lax.Precision.HIGHEST) + fcb_ref[...]


def _blockdiag(a):
    H, C = a.shape
    eye = jnp.eye(H, dtype=a.dtype)
    return (a[:, :, None] * eye[:, None, :]).reshape(H * C, H)


def kernel(x, edge_index, W1, a_src1, a_dst1, b1, W2, a_src2, a_dst2, b2,
           fcW, fcb):
    pad = E_PAD - E
    src = jnp.concatenate([edge_index[0], jnp.zeros((pad,), jnp.int32)])
    dst = jnp.concatenate([edge_index[1], jnp.full((pad,), N, jnp.int32)])

    asrc1 = _blockdiag(a_src1)
    adst1 = _blockdiag(a_dst1)
    asrc2 = _blockdiag(a_src2)
    adst2 = _blockdiag(a_dst2)

    s1, ad1, gmax1 = pl.pallas_call(
        _prep1_body,
        grid=(_GRID,),
        in_specs=[
            pl.BlockSpec((_BLK, 1), lambda i: (i, 0)),
            pl.BlockSpec((1, 32), _full2),
            pl.BlockSpec((32, 4), _full2),
            pl.BlockSpec((32, 4), _full2),
        ],
        out_specs=[
            pl.BlockSpec((_BLK, SW), lambda i: (i, 0)),
            pl.BlockSpec((_BLK, 16), lambda i: (i, 0)),
            pl.BlockSpec((8, 128), _full2),
        ],
        out_shape=[
            jax.ShapeDtypeStruct((N, SW), jnp.float32),
            jax.ShapeDtypeStruct((N, 16), jnp.float32),
            jax.ShapeDtypeStruct((8, 128), jnp.float32),
        ],
        scratch_shapes=[pltpu.VMEM((8, 128), jnp.float32)],
    )(x, W1, asrc1, adst1)

    p1 = pl.pallas_call(
        functools.partial(_mhat_body, 4),
        grid=(_GRID,),
        in_specs=[
            pl.BlockSpec((_BLK, 16), lambda i: (i, 0)),
            pl.BlockSpec((8, 128), _full2),
        ],
        out_specs=pl.BlockSpec((_BLK, PW), lambda i: (i, 0)),
        out_shape=jax.ShapeDtypeStruct((N, PW), jnp.float32),
    )(ad1, gmax1)
    p1 = jnp.concatenate([p1, jnp.zeros((8, PW), jnp.float32)], axis=0)

    acc1 = _sc_l1(src, dst, s1, p1)

    s2a, s2b, ad2, gmax2 = pl.pallas_call(
        _prep2_body,
        grid=(_GRID,),
        in_specs=[
            pl.BlockSpec((_BLK, AW), lambda i: (i, 0)),
            pl.BlockSpec((1, 32), _full2),
            pl.BlockSpec((32, 64), _full2),
            pl.BlockSpec((64, 8), _full2),
            pl.BlockSpec((64, 8), _full2),
        ],
        out_specs=[
            pl.BlockSpec((_BLK, SW), lambda i: (i, 0)),
            pl.BlockSpec((_BLK, SW), lambda i: (i, 0)),
            pl.BlockSpec((_BLK, 16), lambda i: (i, 0)),
            pl.BlockSpec((8, 128), _full2),
        ],
        out_shape=[
            jax.ShapeDtypeStruct((N, SW), jnp.float32),
            jax.ShapeDtypeStruct((N, SW), jnp.float32),
            jax.ShapeDtypeStruct((N, 16), jnp.float32),
            jax.ShapeDtypeStruct((8, 128), jnp.float32),
        ],
        scratch_shapes=[pltpu.VMEM((8, 128), jnp.float32)],
    )(acc1, b1.reshape(1, 32), W2, asrc2, adst2)

    p2 = pl.pallas_call(
        functools.partial(_mhat_body, 8),
        grid=(_GRID,),
        in_specs=[
            pl.BlockSpec((_BLK, 16), lambda i: (i, 0)),
            pl.BlockSpec((8, 128), _full2),
        ],
        out_specs=pl.BlockSpec((_BLK, PW), lambda i: (i, 0)),
        out_shape=jax.ShapeDtypeStruct((N, PW), jnp.float32),
    )(ad2, gmax2)
    p2 = jnp.concatenate([p2, jnp.zeros((8, PW), jnp.float32)], axis=0)

    acc2a = _sc_l2a(src, dst, s2a, p2)
    acc2b = _sc_l2b(src, dst, s2b, p2)

    y = pl.pallas_call(
        _post2_body,
        grid=(_GRID,),
        in_specs=[
            pl.BlockSpec((_BLK, AW), lambda i: (i, 0)),
            pl.BlockSpec((_BLK, AW), lambda i: (i, 0)),
            pl.BlockSpec((1, 64), _full2),
            pl.BlockSpec((64, 1), _full2),
            pl.BlockSpec((1, 1), _full2),
        ],
        out_specs=pl.BlockSpec((_BLK, 1), lambda i: (i, 0)),
        out_shape=jax.ShapeDtypeStruct((N, 1), jnp.float32),
    )(acc2a, acc2b, b2.reshape(1, 64), fcW, fcb.reshape(1, 1))
    return y


def _full2(i):
    return (0, 0)
